# baseline (speedup 1.0000x reference)
import jax
import jax.numpy as jnp
from jax import lax
from jax.experimental import pallas as pl
from jax.experimental.pallas import tpu as pltpu

T = 4096
D_IN = 1024
D_OUT = 1024
E = 8
K = 2
BT = 2048


def _moe_body(x_ref, xf_ref, wg_ref, bg_ref, we_ref, be_ref, out_ref, ew_ref):
    e = pl.program_id(1)
    x = x_ref[...]

    logits = jnp.dot(xf_ref[...], wg_ref[...], preferred_element_type=jnp.float32)
    logits = logits + bg_ref[...]
    iota = lax.broadcasted_iota(jnp.int32, (BT, E), 1)
    m1 = jnp.max(logits, axis=-1, keepdims=True)
    i1 = jnp.min(jnp.where(logits == m1, iota, E), axis=-1, keepdims=True)
    mask1 = iota == i1
    l2 = jnp.where(mask1, -jnp.inf, logits)
    m2 = jnp.max(l2, axis=-1, keepdims=True)
    i2 = jnp.min(jnp.where(l2 == m2, iota, E), axis=-1, keepdims=True)
    mask2 = iota == i2
    b = jnp.exp(m2 - m1)
    denom = 1.0 + b
    w = jnp.where(mask1, 1.0 / denom, 0.0) + jnp.where(mask2, b / denom, 0.0)

    @pl.when(e == 0)
    def _():
        ew_ref[...] = w

    w_col = jnp.sum(jnp.where(iota == e, w, 0.0), axis=-1, keepdims=True)
    contrib = (jnp.dot(x, we_ref[0], preferred_element_type=jnp.float32)
               + be_ref[0]) * w_col

    @pl.when(e == 0)
    def _():
        out_ref[...] = contrib

    @pl.when(e > 0)
    def _():
        out_ref[...] += contrib


def kernel(x, Wg, bg, We, be):
    bg2 = bg.reshape(1, E)
    be3 = be.reshape(E, 1, D_OUT)
    x_bf = x.astype(jnp.bfloat16)
    we_bf = We.astype(jnp.bfloat16)
    grid = (T // BT, E)
    out, ew = pl.pallas_call(
        _moe_body,
        grid=grid,
        in_specs=[
            pl.BlockSpec((BT, D_IN), lambda i, e: (i, 0)),
            pl.BlockSpec((BT, D_IN), lambda i, e: (i, 0)),
            pl.BlockSpec((D_IN, E), lambda i, e: (0, 0)),
            pl.BlockSpec((1, E), lambda i, e: (0, 0)),
            pl.BlockSpec((1, D_IN, D_OUT), lambda i, e: (e, 0, 0)),
            pl.BlockSpec((1, 1, D_OUT), lambda i, e: (e, 0, 0)),
        ],
        out_specs=[
            pl.BlockSpec((BT, D_OUT), lambda i, e: (i, 0)),
            pl.BlockSpec((BT, E), lambda i, e: (i, 0)),
        ],
        out_shape=[
            jax.ShapeDtypeStruct((T, D_OUT), jnp.float32),
            jax.ShapeDtypeStruct((T, E), jnp.float32),
        ],
        compiler_params=pltpu.CompilerParams(
            dimension_semantics=("parallel", "arbitrary"),
        ),
    )(x_bf, x, Wg, bg2, we_bf, be3)
    return out, ew


# speedup vs baseline: 1.0445x; 1.0445x over previous
import functools

import jax
import jax.numpy as jnp
from jax import lax
from jax.experimental import pallas as pl
from jax.experimental.pallas import tpu as pltpu
from jax.experimental.pallas import tpu_sc as plsc

T = 4096
D_IN = 1024
D_OUT = 1024
E = 8
K = 2
BT = 2048

NC = 2
NS = 16
L = 16
TPS = T // (NC * NS)
PPS = TPS * K

_SC_PARAMS = pltpu.CompilerParams(needs_layout_passes=False)


def _moe_body(x_ref, wg_ref, bg_ref, we_ref, be_ref,
              out_ref, tik_ref, gwk_ref):
    e = pl.program_id(1)
    x = x_ref[...]

    logits = jnp.dot(x, wg_ref[...], preferred_element_type=jnp.float32)
    logits = logits + bg_ref[...]
    iota = lax.broadcasted_iota(jnp.int32, (BT, E), 1)
    m1 = jnp.max(logits, axis=-1, keepdims=True)
    i1 = jnp.min(jnp.where(logits == m1, iota, E), axis=-1, keepdims=True)
    mask1 = iota == i1
    l2 = jnp.where(mask1, -jnp.inf, logits)
    m2 = jnp.max(l2, axis=-1, keepdims=True)
    i2 = jnp.min(jnp.where(l2 == m2, iota, E), axis=-1, keepdims=True)
    mask2 = iota == i2
    b = jnp.exp(m2 - m1)
    denom = 1.0 + b
    w1 = 1.0 / denom
    w2 = b / denom
    w = jnp.where(mask1, w1, 0.0) + jnp.where(mask2, w2, 0.0)

    @pl.when(e == 0)
    def _():
        tik_ref[...] = jnp.concatenate([i1, i2], axis=1)
        gwk_ref[...] = jnp.concatenate([w1, w2], axis=1)

    w_col = jnp.sum(jnp.where(iota == e, w, 0.0), axis=-1, keepdims=True)
    contrib = (jnp.dot(x, we_ref[0], preferred_element_type=jnp.float32)
               + be_ref[0]) * w_col

    @pl.when(e == 0)
    def _():
        out_ref[...] = contrib

    @pl.when(e > 0)
    def _():
        out_ref[...] += contrib


def _moe_dense(x, Wg, bg, We, be):
    return pl.pallas_call(
        _moe_body,
        grid=(T // BT, E),
        in_specs=[
            pl.BlockSpec((BT, D_IN), lambda i, e: (i, 0)),
            pl.BlockSpec((D_IN, E), lambda i, e: (0, 0)),
            pl.BlockSpec((1, E), lambda i, e: (0, 0)),
            pl.BlockSpec((1, D_IN, D_OUT), lambda i, e: (e, 0, 0)),
            pl.BlockSpec((1, 1, D_OUT), lambda i, e: (e, 0, 0)),
        ],
        out_specs=[
            pl.BlockSpec((BT, D_OUT), lambda i, e: (i, 0)),
            pl.BlockSpec((BT, K), lambda i, e: (i, 0)),
            pl.BlockSpec((BT, K), lambda i, e: (i, 0)),
        ],
        out_shape=[
            jax.ShapeDtypeStruct((T, D_OUT), jnp.float32),
            jax.ShapeDtypeStruct((T, K), jnp.int32),
            jax.ShapeDtypeStruct((T, K), jnp.float32),
        ],
        compiler_params=pltpu.CompilerParams(
            dimension_semantics=("parallel", "arbitrary"),
        ),
    )(x, Wg, bg.reshape(1, E), We, be.reshape(E, 1, D_OUT))


def _ew_body(tik_hbm, gwk_hbm, ew_hbm, idx_v, gw_v, ew_v):
    c = lax.axis_index("c")
    s = lax.axis_index("s")
    wid = c * NS + s
    pair_base = wid * PPS

    pltpu.sync_copy(tik_hbm.at[pl.ds(pair_base, PPS)], idx_v)
    pltpu.sync_copy(gwk_hbm.at[pl.ds(pair_base, PPS)], gw_v)

    zf = jnp.zeros((L,), jnp.float32)
    for i in range(TPS * E // L):
        ew_v[pl.ds(i * L, L)] = zf
    lanes = lax.iota(jnp.int32, L)
    for j in range(PPS // L):
        tloc = (jnp.full((L,), j * L, jnp.int32) + lanes) >> 1
        tgt = tloc * E + jnp.clip(idx_v[pl.ds(j * L, L)], 0, E - 1)
        plsc.store_scatter(ew_v, [tgt], gw_v[pl.ds(j * L, L)])
    pltpu.sync_copy(ew_v, ew_hbm.at[pl.ds(wid * TPS * E, TPS * E)])


def _ew_scatter(tik_flat, gwk_flat):
    mesh = plsc.VectorSubcoreMesh(core_axis_name="c", subcore_axis_name="s")
    f = functools.partial(
        pl.kernel,
        mesh=mesh,
        compiler_params=_SC_PARAMS,
        out_type=jax.ShapeDtypeStruct((T * E,), jnp.float32),
        scratch_types=[
            pltpu.VMEM((PPS,), jnp.int32),
            pltpu.VMEM((PPS,), jnp.float32),
            pltpu.VMEM((TPS * E,), jnp.float32),
        ],
    )(_ew_body)
    return f(tik_flat, gwk_flat)


def kernel(x, Wg, bg, We, be):
    out, tik, gwk = _moe_dense(x, Wg, bg, We, be)
    ew = _ew_scatter(tik.reshape(T * K), gwk.reshape(T * K))
    return out, ew.reshape(T, E)


# speedup vs baseline: 1.3281x; 1.2715x over previous
import functools

import jax
import jax.numpy as jnp
from jax import lax
from jax.experimental import pallas as pl
from jax.experimental.pallas import tpu as pltpu
from jax.experimental.pallas import tpu_sc as plsc

T = 4096
D_IN = 1024
D_OUT = 1024
E = 8
K = 2
BT = 2048

NC = 2
NS = 16
L = 16
TPS = T // (NC * NS)
PPS = TPS * K

_SC_PARAMS = pltpu.CompilerParams(needs_layout_passes=False)


BTR = 1024


def _router_body(x_ref, wg_ref, bg_ref, tik_ref, gwk_ref):
    x = x_ref[...]
    logits = jnp.dot(x, wg_ref[...], preferred_element_type=jnp.float32)
    logits = logits + bg_ref[...]
    iota = lax.broadcasted_iota(jnp.int32, (BTR, E), 1)
    m1 = jnp.max(logits, axis=-1, keepdims=True)
    i1 = jnp.min(jnp.where(logits == m1, iota, E), axis=-1, keepdims=True)
    mask1 = iota == i1
    l2 = jnp.where(mask1, -jnp.inf, logits)
    m2 = jnp.max(l2, axis=-1, keepdims=True)
    i2 = jnp.min(jnp.where(l2 == m2, iota, E), axis=-1, keepdims=True)
    b = jnp.exp(m2 - m1)
    denom = 1.0 + b
    tik_ref[...] = jnp.concatenate([i1, i2], axis=1)
    gwk_ref[...] = jnp.concatenate([1.0 / denom, b / denom], axis=1)


def _router(x, Wg, bg):
    return pl.pallas_call(
        _router_body,
        grid=(T // BTR,),
        in_specs=[
            pl.BlockSpec((BTR, D_IN), lambda i: (i, 0)),
            pl.BlockSpec((D_IN, E), lambda i: (0, 0)),
            pl.BlockSpec((1, E), lambda i: (0, 0)),
        ],
        out_specs=[
            pl.BlockSpec((BTR, K), lambda i: (i, 0)),
            pl.BlockSpec((BTR, K), lambda i: (i, 0)),
        ],
        out_shape=[
            jax.ShapeDtypeStruct((T, K), jnp.int32),
            jax.ShapeDtypeStruct((T, K), jnp.float32),
        ],
        compiler_params=pltpu.CompilerParams(
            dimension_semantics=("parallel",),
        ),
    )(x, Wg, bg.reshape(1, E))


def _moe_body(x_ref, tik_ref, gwk_ref, we_ref, be_ref, out_ref):
    e = pl.program_id(1)
    x = x_ref[...]
    iota = lax.broadcasted_iota(jnp.int32, (BT, E), 1)
    tik = tik_ref[...]
    gwk = gwk_ref[...]
    w = (jnp.where(iota == tik[:, 0:1], gwk[:, 0:1], 0.0)
         + jnp.where(iota == tik[:, 1:2], gwk[:, 1:2], 0.0))
    w_col = jnp.sum(jnp.where(iota == e, w, 0.0), axis=-1, keepdims=True)
    contrib = (jnp.dot(x, we_ref[0], preferred_element_type=jnp.float32)
               + be_ref[0]) * w_col

    @pl.when(e == 0)
    def _():
        out_ref[...] = contrib

    @pl.when(e > 0)
    def _():
        out_ref[...] += contrib


def _moe_dense(x, tik, gwk, We, be):
    return pl.pallas_call(
        _moe_body,
        grid=(T // BT, E),
        in_specs=[
            pl.BlockSpec((BT, D_IN), lambda i, e: (i, 0)),
            pl.BlockSpec((BT, K), lambda i, e: (i, 0)),
            pl.BlockSpec((BT, K), lambda i, e: (i, 0)),
            pl.BlockSpec((1, D_IN, D_OUT), lambda i, e: (e, 0, 0)),
            pl.BlockSpec((1, 1, D_OUT), lambda i, e: (e, 0, 0)),
        ],
        out_specs=pl.BlockSpec((BT, D_OUT), lambda i, e: (i, 0)),
        out_shape=jax.ShapeDtypeStruct((T, D_OUT), jnp.float32),
        compiler_params=pltpu.CompilerParams(
            dimension_semantics=("parallel", "arbitrary"),
        ),
    )(x, tik, gwk, We, be.reshape(E, 1, D_OUT))


def _ew_body(tik_hbm, gwk_hbm, ew_hbm, idx_v, gw_v, ew_v):
    c = lax.axis_index("c")
    s = lax.axis_index("s")
    wid = c * NS + s
    pair_base = wid * PPS

    pltpu.sync_copy(tik_hbm.at[pl.ds(pair_base, PPS)], idx_v)
    pltpu.sync_copy(gwk_hbm.at[pl.ds(pair_base, PPS)], gw_v)

    zf = jnp.zeros((L,), jnp.float32)
    for i in range(TPS * E // L):
        ew_v[pl.ds(i * L, L)] = zf
    lanes = lax.iota(jnp.int32, L)
    for j in range(PPS // L):
        tloc = (jnp.full((L,), j * L, jnp.int32) + lanes) >> 1
        tgt = tloc * E + jnp.clip(idx_v[pl.ds(j * L, L)], 0, E - 1)
        plsc.store_scatter(ew_v, [tgt], gw_v[pl.ds(j * L, L)])
    pltpu.sync_copy(ew_v, ew_hbm.at[pl.ds(wid * TPS * E, TPS * E)])


def _ew_scatter(tik_flat, gwk_flat):
    mesh = plsc.VectorSubcoreMesh(core_axis_name="c", subcore_axis_name="s")
    f = functools.partial(
        pl.kernel,
        mesh=mesh,
        compiler_params=_SC_PARAMS,
        out_type=jax.ShapeDtypeStruct((T * E,), jnp.float32),
        scratch_types=[
            pltpu.VMEM((PPS,), jnp.int32),
            pltpu.VMEM((PPS,), jnp.float32),
            pltpu.VMEM((TPS * E,), jnp.float32),
        ],
    )(_ew_body)
    return f(tik_flat, gwk_flat)


def kernel(x, Wg, bg, We, be):
    tik, gwk = _router(x, Wg, bg)
    out = _moe_dense(x, tik, gwk, We, be)
    ew = _ew_scatter(tik.reshape(T * K), gwk.reshape(T * K))
    return out, ew.reshape(T, E)
